# baseline (speedup 1.0000x reference)
import functools

import jax
import jax.numpy as jnp
from jax import lax
from jax.experimental import pallas as pl
from jax.experimental.pallas import tpu as pltpu
from jax.experimental.pallas import tpu_sc as plsc

_N = 32768
_D = 512
_B = 16
_C = 104
_NW = 32
_NR = 2 * _NW
_CHUNK = _N // _NW
_ROWS = 64
_NCH = _CHUNK // _ROWS
_NJ = _D // 16
_NEG = -1.0e30

_HIGH = jax.lax.Precision.HIGHEST


def _wid():
    info = plsc.get_sparse_core_info()
    return lax.axis_index("s") * info.num_cores + lax.axis_index("c")


def _hsum16(v):
    iota = lax.iota(jnp.int32, 16)
    for st in (8, 4, 2, 1):
        idx = jnp.bitwise_xor(iota, st)
        v = v + v.at[idx].get(mode="promise_in_bounds", unique_indices=True)
    return v


def _make_k1():
    mesh = plsc.VectorSubcoreMesh(core_axis_name="c", subcore_axis_name="s")

    @functools.partial(
        pl.kernel,
        mesh=mesh,
        out_type=jax.ShapeDtypeStruct((_NR, _D), jnp.float32),
        scratch_types=[
            pltpu.VMEM((2, _ROWS, _D), jnp.float32),
            pltpu.VMEM((2, _D), jnp.float32),
            pltpu.VMEM((16,), jnp.int32),
            pltpu.SemaphoreType.DMA,
            pltpu.SemaphoreType.DMA,
        ],
    )
    def k1(x_hbm, sched_hbm, out_hbm, xbuf, acc, schedv, sem0, sem1):
        wid = _wid()
        base = wid * _CHUNK
        pltpu.sync_copy(sched_hbm.at[wid], schedv)
        len0 = schedv[...][0]

        zeros = jnp.zeros((16,), jnp.float32)
        for run in range(2):
            for j in range(_NJ):
                acc[run, pl.ds(16 * j, 16)] = zeros

        sems = (sem0, sem1)

        def dma(c, buf):
            return pltpu.make_async_copy(
                x_hbm.at[pl.ds(base + c * _ROWS, _ROWS), :],
                xbuf.at[buf],
                sems[buf],
            )

        def row_sum(lo, hi, buf, unroll):
            zero32 = tuple(jnp.zeros((16,), jnp.float32) for _ in range(_NJ))

            def body(r, carry, buf=buf):
                return tuple(
                    carry[j] + xbuf[buf, r, pl.ds(16 * j, 16)]
                    for j in range(_NJ)
                )

            return plsc.parallel_loop(lo, hi, unroll=unroll,
                                      carry=zero32)(body)

        def row_sum2(buf):
            zero32 = tuple(jnp.zeros((16,), jnp.float32) for _ in range(_NJ))

            def body(r2, carry, buf=buf):
                return tuple(
                    carry[j] + (xbuf[buf, 2 * r2, pl.ds(16 * j, 16)]
                                + xbuf[buf, 2 * r2 + 1, pl.ds(16 * j, 16)])
                    for j in range(_NJ)
                )

            return plsc.parallel_loop(0, _ROWS // 2, unroll=2,
                                      carry=zero32)(body)

        def bump(run, vals):
            for j in range(_NJ):
                acc[run, pl.ds(16 * j, 16)] = acc[run, pl.ds(16 * j, 16)] + vals[j]

        def process(c, buf):
            s_split = jnp.clip(len0 - c * _ROWS, 0, _ROWS)

            def all_a():
                bump(0, row_sum2(buf))

            def all_b():
                bump(1, row_sum2(buf))

            def mixed():
                total = row_sum2(buf)
                pref = row_sum(0, s_split, buf, 1)
                bump(0, pref)
                bump(1, tuple(t - p for t, p in zip(total, pref)))

            lax.cond(
                s_split == _ROWS, all_a,
                lambda: lax.cond(s_split == 0, all_b, mixed))

        dma(0, 0).start()

        def gbody(g, carry):
            c0 = 2 * g
            dma(c0 + 1, 1).start()
            dma(c0, 0).wait()
            process(c0, 0)

            @pl.when(c0 + 2 < _NCH)
            def _():
                dma(c0 + 2, 0).start()

            dma(c0 + 1, 1).wait()
            process(c0 + 1, 1)
            return carry

        lax.fori_loop(0, _NCH // 2, gbody, 0)

        pltpu.sync_copy(acc, out_hbm.at[pl.ds(2 * wid, 2), :])

    return k1


def _make_k2():
    mesh = plsc.VectorSubcoreMesh(core_axis_name="c", subcore_axis_name="s")

    @functools.partial(
        pl.kernel,
        mesh=mesh,
        out_type=(
            jax.ShapeDtypeStruct((_NR, 16), jnp.float32),
            jax.ShapeDtypeStruct((_NR, 16), jnp.float32),
            jax.ShapeDtypeStruct((_NR, _D), jnp.float32),
        ),
        scratch_types=[
            pltpu.VMEM((2, _ROWS, _D), jnp.float32),
            pltpu.VMEM((_ROWS, 16), jnp.float32),
            pltpu.VMEM((_B, _D), jnp.float32),
            pltpu.VMEM((2, _D), jnp.float32),
            pltpu.VMEM((2, 16), jnp.float32),
            pltpu.VMEM((2, 16), jnp.float32),
            pltpu.VMEM((16,), jnp.int32),
            pltpu.SemaphoreType.DMA,
            pltpu.SemaphoreType.DMA,
            pltpu.SemaphoreType.DMA,
        ],
    )
    def k2(x_hbm, h_hbm, sched_hbm, m_hbm, d_hbm, c_hbm,
           xbuf, sbuf, hbuf, cacc, dacc, mvv, schedv, sem0, sem1, semh):
        wid = _wid()
        base = wid * _CHUNK
        pltpu.sync_copy(sched_hbm.at[wid], schedv)
        len0 = schedv[...][0]
        seg0 = schedv[...][1]
        pltpu.make_async_copy(h_hbm, hbuf, semh).start()

        neg = jnp.full((16,), _NEG, jnp.float32)
        zeros = jnp.zeros((16,), jnp.float32)
        for run in range(2):
            mvv[run] = neg
            dacc[run] = zeros
            for j in range(_NJ):
                cacc[run, pl.ds(16 * j, 16)] = zeros

        sems = (sem0, sem1)

        def xdma(c, buf):
            return pltpu.make_async_copy(
                x_hbm.at[pl.ds(base + c * _ROWS, _ROWS), :],
                xbuf.at[buf],
                sems[buf],
            )

        def _dot_tree(r, hs, buf):
            parts = [xbuf[buf, r, pl.ds(16 * j, 16)] * hs[j]
                     for j in range(_NJ)]
            while len(parts) > 1:
                nxt = [parts[i] + parts[i + 1]
                       for i in range(0, len(parts) - 1, 2)]
                if len(parts) % 2:
                    nxt.append(parts[-1])
                parts = nxt
            return _hsum16(parts[0])

        def do_run(runi, lo, hi, seg, buf, static2):
            segc = jnp.minimum(seg, _B - 1)
            hs = tuple(hbuf[segc, pl.ds(16 * j, 16)] for j in range(_NJ))

            def dbody(r, smax, hs=hs, buf=buf):
                svec = _dot_tree(r, hs, buf)
                sbuf[r] = svec
                return jnp.maximum(smax, svec)

            def dbody2(r2, smax, hs=hs, buf=buf):
                s0 = _dot_tree(2 * r2, hs, buf)
                s1 = _dot_tree(2 * r2 + 1, hs, buf)
                sbuf[2 * r2] = s0
                sbuf[2 * r2 + 1] = s1
                return jnp.maximum(smax, jnp.maximum(s0, s1))

            if static2:
                smax = plsc.parallel_loop(0, _ROWS // 2, unroll=2,
                                          carry=neg)(dbody2)
            else:
                smax = plsc.parallel_loop(lo, hi, unroll=1, carry=neg)(dbody)

            m_old = mvv[runi]
            m_new = jnp.maximum(m_old, smax)
            gam = jnp.exp(m_old - m_new)
            mvv[runi] = m_new
            init = (dacc[runi] * gam,) + tuple(
                cacc[runi, pl.ds(16 * j, 16)] * gam for j in range(_NJ))

            def wbody(r, carry, buf=buf, m_new=m_new):
                d = carry[0]
                cs = carry[1:]
                w = jnp.exp(sbuf[r] - m_new)
                ncs = tuple(
                    cs[j] + w * xbuf[buf, r, pl.ds(16 * j, 16)]
                    for j in range(_NJ)
                )
                return (d + w,) + ncs

            def wbody2(r2, carry, buf=buf, m_new=m_new):
                d = carry[0]
                cs = carry[1:]
                w0 = jnp.exp(sbuf[2 * r2] - m_new)
                w1 = jnp.exp(sbuf[2 * r2 + 1] - m_new)
                ncs = tuple(
                    cs[j] + (w0 * xbuf[buf, 2 * r2, pl.ds(16 * j, 16)]
                             + w1 * xbuf[buf, 2 * r2 + 1, pl.ds(16 * j, 16)])
                    for j in range(_NJ)
                )
                return (d + (w0 + w1),) + ncs

            if static2:
                res = plsc.parallel_loop(0, _ROWS // 2, unroll=2,
                                         carry=init)(wbody2)
            else:
                res = plsc.parallel_loop(lo, hi, unroll=1, carry=init)(wbody)
            dacc[runi] = res[0]
            for j in range(_NJ):
                cacc[runi, pl.ds(16 * j, 16)] = res[j + 1]

        def process(c, buf):
            s_split = jnp.clip(len0 - c * _ROWS, 0, _ROWS)

            def all_a():
                do_run(0, 0, _ROWS, seg0, buf, True)

            def all_b():
                do_run(1, 0, _ROWS, seg0 + 1, buf, True)

            def mixed():
                do_run(0, 0, s_split, seg0, buf, False)
                do_run(1, s_split, _ROWS, seg0 + 1, buf, False)

            lax.cond(
                s_split == _ROWS, all_a,
                lambda: lax.cond(s_split == 0, all_b, mixed))

        xdma(0, 0).start()
        pltpu.make_async_copy(h_hbm, hbuf, semh).wait()

        def gbody(g, carry):
            c0 = 2 * g
            xdma(c0 + 1, 1).start()
            xdma(c0, 0).wait()
            process(c0, 0)

            @pl.when(c0 + 2 < _NCH)
            def _():
                xdma(c0 + 2, 0).start()

            xdma(c0 + 1, 1).wait()
            process(c0 + 1, 1)
            return carry

        lax.fori_loop(0, _NCH // 2, gbody, 0)

        pltpu.sync_copy(mvv, m_hbm.at[pl.ds(2 * wid, 2), :])
        pltpu.sync_copy(dacc, d_hbm.at[pl.ds(2 * wid, 2), :])
        pltpu.sync_copy(cacc, c_hbm.at[pl.ds(2 * wid, 2), :])

    return k2


def _tc_prep(p, r_mean, w_attn):
    def body(p_ref, r_ref, w_ref, o_ref):
        init = jnp.dot(r_ref[...], p_ref[...], precision=_HIGH,
                       preferred_element_type=jnp.float32)
        o_ref[...] = lax.dot_general(
            init, w_ref[...], (((1,), (1,)), ((), ())), precision=_HIGH,
            preferred_element_type=jnp.float32)

    return pl.pallas_call(
        body, out_shape=jax.ShapeDtypeStruct((_B, _D), jnp.float32),
    )(p, r_mean, w_attn)


def _tc_combine(m, d, cn, rh, w_lin, b2):
    def body(m_ref, d_ref, cn_ref, rh_ref, wl_ref, b_ref, o_ref):
        ones_b = jnp.ones((_B, 1), jnp.float32)
        mt = lax.dot_general(ones_b, m_ref[:, 0:1], (((1,), (1,)), ((), ())),
                             precision=_HIGH,
                             preferred_element_type=jnp.float32)
        dt = lax.dot_general(ones_b, d_ref[:, 0:1], (((1,), (1,)), ((), ())),
                             precision=_HIGH,
                             preferred_element_type=jnp.float32)
        rh_v = rh_ref[...]
        valid = rh_v > 0
        mseg = jnp.max(jnp.where(valid, mt, _NEG), axis=1, keepdims=True)
        alpha = jnp.exp(jnp.where(valid, mt - mseg, _NEG))
        denom = jnp.sum(alpha * dt, axis=1, keepdims=True)
        ctx = jnp.dot(alpha, cn_ref[...], precision=_HIGH,
                      preferred_element_type=jnp.float32) / denom
        out = lax.dot_general(ctx, wl_ref[...], (((1,), (1,)), ((), ())),
                              precision=_HIGH,
                              preferred_element_type=jnp.float32)
        o_ref[...] = out + b_ref[0:1, :]

    return pl.pallas_call(
        body, out_shape=jax.ShapeDtypeStruct((_B, _C), jnp.float32),
    )(m, d, cn, rh, w_lin, b2)


_k1 = _make_k1()
_k2 = _make_k2()


def kernel(encoded_paths, contexts_per_label, W_attn, W_lin, b_lin):
    x = encoded_paths
    counts = contexts_per_label.astype(jnp.int32)
    off = jnp.cumsum(counts)
    lo = jnp.arange(_NW, dtype=jnp.int32) * _CHUNK
    seg0 = jnp.searchsorted(off, lo, side="right").astype(jnp.int32)
    end0 = jnp.take(off, seg0)
    len0 = jnp.minimum(end0 - lo, _CHUNK)
    seg1 = jnp.minimum(seg0 + 1, _B - 1)
    run_seg = jnp.stack([seg0, seg1], axis=1).reshape(_NR)
    rh = (run_seg[None, :] == jnp.arange(_B, dtype=jnp.int32)[:, None]
          ).astype(jnp.float32)
    r_mean = rh / counts.astype(jnp.float32)[:, None]
    sched = (jnp.zeros((_NW, 16), jnp.int32)
             .at[:, 0].set(len0)
             .at[:, 1].set(seg0))
    b2 = jnp.broadcast_to(b_lin, (8, _C))

    p = _k1(x, sched)
    hidden = _tc_prep(p, r_mean, W_attn)
    m, d, c = _k2(x, hidden, sched)
    return _tc_combine(m, d, c, rh, W_lin, b2)


# speedup vs baseline: 1.0208x; 1.0208x over previous
import functools

import jax
import jax.numpy as jnp
from jax import lax
from jax.experimental import pallas as pl
from jax.experimental.pallas import tpu as pltpu
from jax.experimental.pallas import tpu_sc as plsc

_N = 32768
_D = 512
_B = 16
_C = 104
_NW = 32
_NR = 2 * _NW
_CHUNK = _N // _NW
_ROWS = 64
_NCH = _CHUNK // _ROWS
_NJ = _D // 16
_NEG = -1.0e30

_HIGH = jax.lax.Precision.HIGHEST


def _wid():
    info = plsc.get_sparse_core_info()
    return lax.axis_index("s") * info.num_cores + lax.axis_index("c")


def _hsum16(v):
    iota = lax.iota(jnp.int32, 16)
    for st in (8, 4, 2, 1):
        idx = jnp.bitwise_xor(iota, st)
        v = v + v.at[idx].get(mode="promise_in_bounds", unique_indices=True)
    return v


def _make_k1():
    mesh = plsc.VectorSubcoreMesh(core_axis_name="c", subcore_axis_name="s")

    @functools.partial(
        pl.kernel,
        mesh=mesh,
        out_type=jax.ShapeDtypeStruct((_NR, _D), jnp.float32),
        scratch_types=[
            pltpu.VMEM((2, _ROWS, _D), jnp.float32),
            pltpu.VMEM((2, _D), jnp.float32),
            pltpu.VMEM((16,), jnp.int32),
            pltpu.SemaphoreType.DMA,
            pltpu.SemaphoreType.DMA,
        ],
    )
    def k1(x_hbm, sched_hbm, out_hbm, xbuf, acc, schedv, sem0, sem1):
        wid = _wid()
        base = wid * _CHUNK
        pltpu.sync_copy(sched_hbm.at[wid], schedv)
        len0 = schedv[...][0]

        zeros = jnp.zeros((16,), jnp.float32)
        for run in range(2):
            for j in range(_NJ):
                acc[run, pl.ds(16 * j, 16)] = zeros

        sems = (sem0, sem1)

        def dma(c, buf):
            return pltpu.make_async_copy(
                x_hbm.at[pl.ds(base + c * _ROWS, _ROWS), :],
                xbuf.at[buf],
                sems[buf],
            )

        def process(c, buf):
            s_split = jnp.clip(len0 - c * _ROWS, 0, _ROWS)
            for run, lo, hi in ((0, 0, s_split), (1, s_split, _ROWS)):
                init = tuple(acc[run, pl.ds(16 * j, 16)] for j in range(_NJ))

                def body(r, carry, buf=buf):
                    return tuple(
                        carry[j] + xbuf[buf, r, pl.ds(16 * j, 16)]
                        for j in range(_NJ)
                    )

                res = plsc.parallel_loop(lo, hi, unroll=4, carry=init)(body)
                for j in range(_NJ):
                    acc[run, pl.ds(16 * j, 16)] = res[j]

        dma(0, 0).start()

        def gbody(g, carry):
            c0 = 2 * g
            dma(c0 + 1, 1).start()
            dma(c0, 0).wait()
            process(c0, 0)

            @pl.when(c0 + 2 < _NCH)
            def _():
                dma(c0 + 2, 0).start()

            dma(c0 + 1, 1).wait()
            process(c0 + 1, 1)
            return carry

        lax.fori_loop(0, _NCH // 2, gbody, 0)

        pltpu.sync_copy(acc, out_hbm.at[pl.ds(2 * wid, 2), :])

    return k1


def _make_k2():
    mesh = plsc.VectorSubcoreMesh(core_axis_name="c", subcore_axis_name="s")

    @functools.partial(
        pl.kernel,
        mesh=mesh,
        out_type=(
            jax.ShapeDtypeStruct((_NR, 16), jnp.float32),
            jax.ShapeDtypeStruct((_NR, 16), jnp.float32),
            jax.ShapeDtypeStruct((_NR, _D), jnp.float32),
        ),
        scratch_types=[
            pltpu.VMEM((2, _ROWS, _D), jnp.float32),
            pltpu.VMEM((_ROWS, 16), jnp.float32),
            pltpu.VMEM((_B, _D), jnp.float32),
            pltpu.VMEM((2, _D), jnp.float32),
            pltpu.VMEM((2, 16), jnp.float32),
            pltpu.VMEM((2, 16), jnp.float32),
            pltpu.VMEM((16,), jnp.int32),
            pltpu.SemaphoreType.DMA,
            pltpu.SemaphoreType.DMA,
            pltpu.SemaphoreType.DMA,
        ],
    )
    def k2(x_hbm, h_hbm, sched_hbm, m_hbm, d_hbm, c_hbm,
           xbuf, sbuf, hbuf, cacc, dacc, mvv, schedv, sem0, sem1, semh):
        wid = _wid()
        base = wid * _CHUNK
        pltpu.sync_copy(sched_hbm.at[wid], schedv)
        len0 = schedv[...][0]
        seg0 = schedv[...][1]
        pltpu.make_async_copy(h_hbm, hbuf, semh).start()

        neg = jnp.full((16,), _NEG, jnp.float32)
        zeros = jnp.zeros((16,), jnp.float32)
        for run in range(2):
            mvv[run] = neg
            dacc[run] = zeros
            for j in range(_NJ):
                cacc[run, pl.ds(16 * j, 16)] = zeros

        sems = (sem0, sem1)

        def xdma(c, buf):
            return pltpu.make_async_copy(
                x_hbm.at[pl.ds(base + c * _ROWS, _ROWS), :],
                xbuf.at[buf],
                sems[buf],
            )

        def _dot_tree(r, hs, buf):
            parts = [xbuf[buf, r, pl.ds(16 * j, 16)] * hs[j]
                     for j in range(_NJ)]
            while len(parts) > 1:
                nxt = [parts[i] + parts[i + 1]
                       for i in range(0, len(parts) - 1, 2)]
                if len(parts) % 2:
                    nxt.append(parts[-1])
                parts = nxt
            return _hsum16(parts[0])

        def do_run(runi, lo, hi, seg, buf):
            segc = jnp.minimum(seg, _B - 1)
            hs = tuple(hbuf[segc, pl.ds(16 * j, 16)] for j in range(_NJ))

            def dbody(r, smax, hs=hs, buf=buf):
                svec = _dot_tree(r, hs, buf)
                sbuf[r] = svec
                return jnp.maximum(smax, svec)

            smax = plsc.parallel_loop(lo, hi, unroll=2, carry=neg)(dbody)

            m_old = mvv[runi]
            m_new = jnp.maximum(m_old, smax)
            gam = jnp.exp(m_old - m_new)
            mvv[runi] = m_new
            init = (dacc[runi] * gam,) + tuple(
                cacc[runi, pl.ds(16 * j, 16)] * gam for j in range(_NJ))

            def wbody(r, carry, buf=buf, m_new=m_new):
                d = carry[0]
                cs = carry[1:]
                w = jnp.exp(sbuf[r] - m_new)
                ncs = tuple(
                    cs[j] + w * xbuf[buf, r, pl.ds(16 * j, 16)]
                    for j in range(_NJ)
                )
                return (d + w,) + ncs

            res = plsc.parallel_loop(lo, hi, unroll=2, carry=init)(wbody)
            dacc[runi] = res[0]
            for j in range(_NJ):
                cacc[runi, pl.ds(16 * j, 16)] = res[j + 1]

        def process(c, buf):
            s_split = jnp.clip(len0 - c * _ROWS, 0, _ROWS)
            do_run(0, 0, s_split, seg0, buf)
            do_run(1, s_split, _ROWS, seg0 + 1, buf)

        xdma(0, 0).start()
        pltpu.make_async_copy(h_hbm, hbuf, semh).wait()

        def gbody(g, carry):
            c0 = 2 * g
            xdma(c0 + 1, 1).start()
            xdma(c0, 0).wait()
            process(c0, 0)

            @pl.when(c0 + 2 < _NCH)
            def _():
                xdma(c0 + 2, 0).start()

            xdma(c0 + 1, 1).wait()
            process(c0 + 1, 1)
            return carry

        lax.fori_loop(0, _NCH // 2, gbody, 0)

        pltpu.sync_copy(mvv, m_hbm.at[pl.ds(2 * wid, 2), :])
        pltpu.sync_copy(dacc, d_hbm.at[pl.ds(2 * wid, 2), :])
        pltpu.sync_copy(cacc, c_hbm.at[pl.ds(2 * wid, 2), :])

    return k2


def _tc_prep(p, r_mean, w_attn):
    def body(p_ref, r_ref, w_ref, o_ref):
        init = jnp.dot(r_ref[...], p_ref[...], precision=_HIGH,
                       preferred_element_type=jnp.float32)
        o_ref[...] = lax.dot_general(
            init, w_ref[...], (((1,), (1,)), ((), ())), precision=_HIGH,
            preferred_element_type=jnp.float32)

    return pl.pallas_call(
        body, out_shape=jax.ShapeDtypeStruct((_B, _D), jnp.float32),
    )(p, r_mean, w_attn)


def _tc_combine(m, d, cn, rh, w_lin, b2):
    def body(m_ref, d_ref, cn_ref, rh_ref, wl_ref, b_ref, o_ref):
        ones_b = jnp.ones((_B, 1), jnp.float32)
        mt = lax.dot_general(ones_b, m_ref[:, 0:1], (((1,), (1,)), ((), ())),
                             precision=_HIGH,
                             preferred_element_type=jnp.float32)
        dt = lax.dot_general(ones_b, d_ref[:, 0:1], (((1,), (1,)), ((), ())),
                             precision=_HIGH,
                             preferred_element_type=jnp.float32)
        rh_v = rh_ref[...]
        valid = rh_v > 0
        mseg = jnp.max(jnp.where(valid, mt, _NEG), axis=1, keepdims=True)
        alpha = jnp.exp(jnp.where(valid, mt - mseg, _NEG))
        denom = jnp.sum(alpha * dt, axis=1, keepdims=True)
        ctx = jnp.dot(alpha, cn_ref[...], precision=_HIGH,
                      preferred_element_type=jnp.float32) / denom
        out = lax.dot_general(ctx, wl_ref[...], (((1,), (1,)), ((), ())),
                              precision=_HIGH,
                              preferred_element_type=jnp.float32)
        o_ref[...] = out + b_ref[0:1, :]

    return pl.pallas_call(
        body, out_shape=jax.ShapeDtypeStruct((_B, _C), jnp.float32),
    )(m, d, cn, rh, w_lin, b2)


_k1 = _make_k1()
_k2 = _make_k2()


def kernel(encoded_paths, contexts_per_label, W_attn, W_lin, b_lin):
    x = encoded_paths
    counts = contexts_per_label.astype(jnp.int32)
    off = jnp.cumsum(counts)
    lo = jnp.arange(_NW, dtype=jnp.int32) * _CHUNK
    seg0 = jnp.searchsorted(off, lo, side="right").astype(jnp.int32)
    end0 = jnp.take(off, seg0)
    len0 = jnp.minimum(end0 - lo, _CHUNK)
    seg1 = jnp.minimum(seg0 + 1, _B - 1)
    run_seg = jnp.stack([seg0, seg1], axis=1).reshape(_NR)
    rh = (run_seg[None, :] == jnp.arange(_B, dtype=jnp.int32)[:, None]
          ).astype(jnp.float32)
    r_mean = rh / counts.astype(jnp.float32)[:, None]
    sched = (jnp.zeros((_NW, 16), jnp.int32)
             .at[:, 0].set(len0)
             .at[:, 1].set(seg0))
    b2 = jnp.broadcast_to(b_lin, (8, _C))

    p = _k1(x, sched)
    hidden = _tc_prep(p, r_mean, W_attn)
    m, d, c = _k2(x, hidden, sched)
    return _tc_combine(m, d, c, rh, W_lin, b2)
